# Initial kernel scaffold; baseline (speedup 1.0000x reference)
import jax
import jax.numpy as jnp
from jax.experimental import pallas as pl


def kernel(feat, segment_ids, W_ih, W_hh, b_ih, b_hh):
    raise NotImplementedError("write your pallas kernel here")


# speedup vs baseline: 6.9061x; 6.9061x over previous
import functools

import jax
import jax.numpy as jnp
from jax import lax
from jax.experimental import pallas as pl
from jax.experimental.pallas import tpu as pltpu
from jax.experimental.pallas import tpu_sc as plsc

NUM_B = 256
N_ITERS = 6
NC = 2
NS = 16
NW = NC * NS
SEGS_PER = NUM_B // NW
CHUNK = 128
CHUNKP = CHUNK + 8
LANES = 16


def _attn_body(feat_hbm, offs_hbm, q_hbm, out_hbm, q_v, offs_v, buf, out_v):
    n_total, d = feat_hbm.shape
    groups = d // LANES
    cid = lax.axis_index("c")
    sid = lax.axis_index("s")
    wid = sid * NC + cid
    b0 = wid * SEGS_PER

    pltpu.sync_copy(offs_hbm.at[pl.ds(b0, 16)], offs_v)
    pltpu.sync_copy(q_hbm.at[pl.ds(b0, SEGS_PER)], q_v)
    ov = offs_v[...]

    for k in range(SEGS_PER):
        rs = ov[k]
        re = ov[k + 1]
        nb = re - rs
        nchunks = (nb + (CHUNK - 1)) >> 7
        qreg = [q_v[k, pl.ds(LANES * j, LANES)] for j in range(groups)]

        def chunk_body(g, carry, rs=rs, re=re, qreg=qreg):
            start = rs + g * CHUNK
            start_c = jnp.minimum((start >> 3) << 3, n_total - CHUNKP)
            start_c = pl.multiple_of(start_c, 8)
            delta = start - start_c
            nrows = jnp.minimum(CHUNK, re - start)
            pltpu.sync_copy(feat_hbm.at[pl.ds(start_c, CHUNKP)], buf)

            def row(i, c2, qreg=qreg, delta=delta):
                m_run = c2[0]
                d_run = c2[1]
                r_acc = c2[2:]
                fv = [buf[i + delta, pl.ds(LANES * j, LANES)] for j in range(groups)]
                acc = fv[0] * qreg[0]
                for j in range(1, groups):
                    acc = acc + fv[j] * qreg[j]
                s = jnp.sum(acc)
                m_new = jnp.maximum(m_run, s)
                sc_vec = jnp.exp(jnp.full((LANES,), m_run - m_new, jnp.float32))
                a_vec = jnp.exp(jnp.full((LANES,), s - m_new, jnp.float32))
                d_new = d_run * sc_vec + a_vec
                r_new = tuple(
                    r_acc[j] * sc_vec + a_vec * fv[j] for j in range(groups)
                )
                return (m_new, d_new) + r_new

            return lax.fori_loop(0, nrows, row, carry)

        init = (jnp.float32(-1e30), jnp.zeros((LANES,), jnp.float32)) + tuple(
            jnp.zeros((LANES,), jnp.float32) for _ in range(groups)
        )
        final = lax.fori_loop(0, nchunks, chunk_body, init)
        d_vec = final[1]
        inv = jnp.where(d_vec > 0.0, 1.0 / d_vec, 0.0)
        for j in range(groups):
            out_v[k, pl.ds(LANES * j, LANES)] = final[2 + j] * inv

    pltpu.sync_copy(out_v, out_hbm.at[pl.ds(b0, SEGS_PER)])


def _lstm_body(h_ref, c_ref, r_ref, a_ref, rw_ref, b_ref, h_out, c_out):
    d = h_ref.shape[1]
    h = h_ref[...]
    c = c_ref[...]
    r = r_ref[...]
    gates = (
        jnp.dot(h, a_ref[...], preferred_element_type=jnp.float32)
        + jnp.dot(r, rw_ref[...], preferred_element_type=jnp.float32)
        + b_ref[...]
    )
    i_g = jax.nn.sigmoid(gates[:, :d])
    f_g = jax.nn.sigmoid(gates[:, d:2 * d])
    g_g = jnp.tanh(gates[:, 2 * d:3 * d])
    o_g = jax.nn.sigmoid(gates[:, 3 * d:])
    c_new = f_g * c + i_g * g_g
    h_new = o_g * jnp.tanh(c_new)
    h_out[...] = h_new
    c_out[...] = c_new


def kernel(feat, segment_ids, W_ih, W_hh, b_ih, b_hh):
    n, d = feat.shape
    b = NUM_B
    groups = d // LANES
    del groups

    offs = jnp.searchsorted(
        segment_ids, jnp.arange(b + 1, dtype=jnp.int32), side="left"
    ).astype(jnp.int32)
    offs = jnp.pad(offs, (0, 264 - (b + 1)), constant_values=n)

    w_ih_t = W_ih.T
    a_w = w_ih_t[:d] + W_hh.T
    r_w = w_ih_t[d:]
    bias = (b_ih + b_hh)[None, :]

    lstm = pl.pallas_call(
        _lstm_body,
        out_shape=(
            jax.ShapeDtypeStruct((b, d), jnp.float32),
            jax.ShapeDtypeStruct((b, d), jnp.float32),
        ),
    )

    mesh = plsc.VectorSubcoreMesh(core_axis_name="c", subcore_axis_name="s")
    attn = functools.partial(
        pl.kernel,
        mesh=mesh,
        compiler_params=pltpu.CompilerParams(needs_layout_passes=False),
        out_type=jax.ShapeDtypeStruct((b, d), jnp.float32),
        scratch_types=[
            pltpu.VMEM((SEGS_PER, d), jnp.float32),
            pltpu.VMEM((16,), jnp.int32),
            pltpu.VMEM((CHUNKP, d), jnp.float32),
            pltpu.VMEM((SEGS_PER, d), jnp.float32),
        ],
    )(_attn_body)

    h = jnp.zeros((b, d), jnp.float32)
    c = jnp.zeros((b, d), jnp.float32)
    readout = jnp.zeros((b, d), jnp.float32)
    for _ in range(N_ITERS):
        h, c = lstm(h, c, readout, a_w, r_w, bias)
        readout = attn(feat, offs, h)
    return jnp.concatenate([h, readout], axis=-1)
